# Initial kernel scaffold; baseline (speedup 1.0000x reference)
import jax
import jax.numpy as jnp
from jax.experimental import pallas as pl


def kernel(embeddings, batch_labels):
    raise NotImplementedError("write your pallas kernel here")


# speedup vs baseline: 32.6712x; 32.6712x over previous
import functools

import jax
import jax.numpy as jnp
from jax.experimental import pallas as pl

_N_NEIGHBORS = 15
_N_BATCHES = 8
_N_CELLS = 4096
_LATENT = 256
_BLOCK = 512
_DIAG = 10000000000.0


def _body(e_ref, et_ref, lab_ref, out_ref):
    i = pl.program_id(0)
    e = e_ref[...]
    et = et_ref[...]
    g = jnp.dot(e, et, preferred_element_type=jnp.float32)
    sq_rows = jnp.sum(e * e, axis=1, keepdims=True)
    sq_cols = jnp.sum(et * et, axis=0, keepdims=True)
    d = sq_rows + sq_cols - 2.0 * g
    col_ids = jax.lax.broadcasted_iota(jnp.int32, d.shape, 1)
    row_ids = jax.lax.broadcasted_iota(jnp.int32, d.shape, 0) + i * _BLOCK
    d = jnp.where(col_ids == row_ids, d + _DIAG, d)

    cur = d
    rm = jnp.min(cur, axis=1, keepdims=True)
    for _ in range(_N_NEIGHBORS - 1):
        cur = jnp.where(cur <= rm, jnp.inf, cur)
        rm = jnp.min(cur, axis=1, keepdims=True)
    thresh = rm

    m = jnp.min(d, axis=1, keepdims=True)
    w = jnp.exp(m - d)
    z = jnp.sum(w, axis=1, keepdims=True)
    wm = jnp.where(d <= thresh, w, 0.0)
    s = jnp.sum(wm, axis=1, keepdims=True)

    lab = lab_ref[...]
    onehot = (lab == jax.lax.broadcasted_iota(
        jnp.int32, (_N_CELLS, _N_BATCHES), 1)).astype(jnp.float32)
    bsum = jnp.dot(wm, onehot, preferred_element_type=jnp.float32)

    p = bsum / (s + 1e-8 * z)
    ent = -jnp.sum(p * jnp.log(p + 1e-8), axis=1)
    nent = ent / (jnp.log(jnp.float32(_N_BATCHES)) + 1e-8)

    @pl.when(i == 0)
    def _():
        out_ref[...] = jnp.zeros((1, 1), jnp.float32)
    out_ref[...] += jnp.sum(nent).reshape(1, 1)


def kernel(embeddings, batch_labels):
    et = embeddings.T
    lab = batch_labels.reshape(_N_CELLS, 1).astype(jnp.int32)
    n_blocks = _N_CELLS // _BLOCK
    acc = pl.pallas_call(
        _body,
        grid=(n_blocks,),
        in_specs=[
            pl.BlockSpec((_BLOCK, _LATENT), lambda i: (i, 0)),
            pl.BlockSpec((_LATENT, _N_CELLS), lambda i: (0, 0)),
            pl.BlockSpec((_N_CELLS, 1), lambda i: (0, 0)),
        ],
        out_specs=pl.BlockSpec((1, 1), lambda i: (0, 0)),
        out_shape=jax.ShapeDtypeStruct((1, 1), jnp.float32),
    )(embeddings, et, lab)
    return -acc[0, 0] / _N_CELLS


# speedup vs baseline: 33.7299x; 1.0324x over previous
import functools

import jax
import jax.numpy as jnp
from jax import lax
from jax.experimental import pallas as pl
from jax.experimental.pallas import tpu as pltpu
from jax.experimental.pallas import tpu_sc as plsc

_N_BATCHES = 8
_N_CELLS = 4096
_LATENT = 256
_BLOCK1 = 256
_BLOCK3 = 512
_DIAG = 10000000000.0
_GROUPS = 128
_GSIZE = 32
_ROWS_PER_W = 128


def _dist_body(e_ref, et_ref, d_ref, gmin_ref):
    i = pl.program_id(0)
    e = e_ref[...]
    et = et_ref[...]
    g = jnp.dot(e, et, preferred_element_type=jnp.float32)
    sq_rows = jnp.sum(e * e, axis=1, keepdims=True)
    sq_cols = jnp.sum(et * et, axis=0, keepdims=True)
    d = sq_rows + sq_cols - 2.0 * g
    col_ids = jax.lax.broadcasted_iota(jnp.int32, d.shape, 1)
    row_ids = jax.lax.broadcasted_iota(jnp.int32, d.shape, 0) + i * _BLOCK1
    d = jnp.where(col_ids == row_ids, d + _DIAG, d)
    d_ref[...] = d
    gmin_ref[...] = jnp.min(
        d.reshape(_BLOCK1 // _GSIZE, _GSIZE, _N_CELLS), axis=1)


def _kv_merge(ka, va, kb, vb):
    rkb = lax.rev(kb, (0,))
    rvb = lax.rev(vb, (0,))
    take_a = ka <= rkb
    kl = jnp.where(take_a, ka, rkb)
    vl = jnp.where(take_a, va, rvb)
    return plsc.sort_key_val(kl, vl)


def _vmerge(a, b):
    lo = jnp.minimum(a, lax.rev(b, (0,)))
    return lax.sort(lo, dimension=0)


_SLAB = 8
_N_SLABS = _ROWS_PER_W // _SLAB


def _sc_select_body(d_hbm, gmin_hbm, out_hbm, gmin_v, rows_v, out_v, sem_d):
    wid = lax.axis_index("s") * 2 + lax.axis_index("c")
    row0 = wid * _ROWS_PER_W
    pltpu.sync_copy(gmin_hbm.at[:, pl.ds(row0, _ROWS_PER_W)], gmin_v)
    iota16 = lax.broadcasted_iota(jnp.int32, (16,), 0)

    pltpu.async_copy(d_hbm.at[pl.ds(row0, _SLAB)], rows_v.at[0], sem_d)

    def slab_body(si, carry):
        buf = si % 2

        @pl.when(si < _N_SLABS - 1)
        def _():
            pltpu.async_copy(
                d_hbm.at[pl.ds(row0 + (si + 1) * _SLAB, _SLAB)],
                rows_v.at[(si + 1) % 2], sem_d)

        pltpu.make_async_copy(
            d_hbm.at[pl.ds(row0, _SLAB)], rows_v.at[buf], sem_d).wait()

        def row_body(j, carry2):
            rr = si * _SLAB + j
            rvec = iota16 * 0 + rr
            ks, vs = [], []
            for c in range(8):
                k = plsc.load_gather(gmin_v, [iota16 + c * 16, rvec])
                v = iota16 + c * 16
                kk, vv = plsc.sort_key_val(k, v)
                ks.append(kk)
                vs.append(vv)
            while len(ks) > 1:
                nk, nv = [], []
                for t in range(0, len(ks), 2):
                    a, b = _kv_merge(ks[t], vs[t], ks[t + 1], vs[t + 1])
                    nk.append(a)
                    nv.append(b)
                ks, vs = nk, nv
            gbase = vs[0] * _GSIZE
            jvec = iota16 * 0 + j
            bvec = iota16 * 0 + buf
            chunks = []
            for c in range(_GSIZE):
                x = plsc.load_gather(rows_v, [bvec, jvec, gbase + c])
                chunks.append(lax.sort(x, dimension=0))
            while len(chunks) > 1:
                chunks = [_vmerge(chunks[t], chunks[t + 1])
                          for t in range(0, len(chunks), 2)]
            out_v[rr, :] = chunks[0]
            return carry2

        lax.fori_loop(0, _SLAB, row_body, 0)
        return carry

    lax.fori_loop(0, _N_SLABS, slab_body, 0)
    pltpu.sync_copy(out_v, out_hbm.at[pl.ds(row0, _ROWS_PER_W)])


def _comb_body(d_ref, b16_ref, lab_ref, out_ref):
    i = pl.program_id(0)
    d = d_ref[...]
    b16 = b16_ref[...]
    m = b16[:, 0:1]
    thresh = b16[:, 14:15]
    w = jnp.exp(m - d)
    z = jnp.sum(w, axis=1, keepdims=True)
    wm = jnp.where(d <= thresh, w, 0.0)
    s = jnp.sum(wm, axis=1, keepdims=True)
    lab = lab_ref[...]
    onehot = (lab == jax.lax.broadcasted_iota(
        jnp.int32, (_N_CELLS, _N_BATCHES), 1)).astype(jnp.float32)
    bsum = jnp.dot(wm, onehot, preferred_element_type=jnp.float32)
    p = bsum / (s + 1e-8 * z)
    ent = -jnp.sum(p * jnp.log(p + 1e-8), axis=1)
    nent = ent / (jnp.log(jnp.float32(_N_BATCHES)) + 1e-8)

    @pl.when(i == 0)
    def _():
        out_ref[...] = jnp.zeros((1, 1), jnp.float32)
    out_ref[...] += jnp.sum(nent).reshape(1, 1)


def kernel(embeddings, batch_labels):
    et = embeddings.T
    lab = batch_labels.reshape(_N_CELLS, 1).astype(jnp.int32)
    d, gmin = pl.pallas_call(
        _dist_body,
        grid=(_N_CELLS // _BLOCK1,),
        in_specs=[
            pl.BlockSpec((_BLOCK1, _LATENT), lambda i: (i, 0)),
            pl.BlockSpec((_LATENT, _N_CELLS), lambda i: (0, 0)),
        ],
        out_specs=[
            pl.BlockSpec((_BLOCK1, _N_CELLS), lambda i: (i, 0)),
            pl.BlockSpec((_BLOCK1 // _GSIZE, _N_CELLS), lambda i: (i, 0)),
        ],
        out_shape=[
            jax.ShapeDtypeStruct((_N_CELLS, _N_CELLS), jnp.float32),
            jax.ShapeDtypeStruct((_GROUPS, _N_CELLS), jnp.float32),
        ],
    )(embeddings, et)

    mesh = plsc.VectorSubcoreMesh(core_axis_name="c", subcore_axis_name="s")
    sc_select = functools.partial(
        pl.kernel,
        mesh=mesh,
        compiler_params=pltpu.CompilerParams(needs_layout_passes=False),
        out_type=jax.ShapeDtypeStruct((_N_CELLS, 16), jnp.float32),
        scratch_types=[
            pltpu.VMEM((_ROWS_PER_W, _GROUPS), jnp.float32),
            pltpu.VMEM((2, _SLAB, _N_CELLS), jnp.float32),
            pltpu.VMEM((_ROWS_PER_W, 16), jnp.float32),
            pltpu.SemaphoreType.DMA,
        ],
    )(_sc_select_body)
    b16 = sc_select(d, gmin)

    acc = pl.pallas_call(
        _comb_body,
        grid=(_N_CELLS // _BLOCK3,),
        in_specs=[
            pl.BlockSpec((_BLOCK3, _N_CELLS), lambda i: (i, 0)),
            pl.BlockSpec((_BLOCK3, 16), lambda i: (i, 0)),
            pl.BlockSpec((_N_CELLS, 1), lambda i: (0, 0)),
        ],
        out_specs=pl.BlockSpec((1, 1), lambda i: (0, 0)),
        out_shape=jax.ShapeDtypeStruct((1, 1), jnp.float32),
    )(d, b16, lab)
    return -acc[0, 0] / _N_CELLS
